# baseline (speedup 1.0000x reference)
import functools

import jax
import jax.numpy as jnp
from jax import lax
from jax.experimental import pallas as pl
from jax.experimental.pallas import tpu as pltpu
from jax.experimental.pallas import tpu_sc as plsc

HB = 64
HBE = 128

NS = 16
L = 16


def _energy_kernel(x_ref, o_ref):
    x = x_ref[...]
    e2 = jnp.sum(x * x, axis=1)[:, None]
    o_ref[...] = lax.bitcast_convert_type(e2, jnp.int32)


def _mask_kernel(thr_ref, e_blk_ref, x_ref, *rest):
    o_ref = rest[-1]
    b = pl.program_id(0)
    thr = thr_ref[b, 0]
    mask = (e_blk_ref[...] >= thr).astype(jnp.float32)
    o_ref[...] = x_ref[...] * mask


def _select_bin(tot_v, nbins, k_b):
    nch = nbins // L
    lane = lax.iota(jnp.int32, L)

    def sum_body(jj, carry):
        acc, jstar, done = carry
        j = nch - 1 - jj
        s = jnp.sum(tot_v[pl.ds(j * L, L)])
        crossing = jnp.logical_and(jnp.logical_not(done), (acc + s) >= k_b)
        jstar = jnp.where(crossing, j, jstar)
        done = jnp.logical_or(done, crossing)
        acc = acc + jnp.where(done, jnp.int32(0), s)
        return acc, jstar, done

    acc, jstar, _ = lax.fori_loop(
        0, nch, sum_body, (jnp.int32(0), jnp.int32(0), jnp.bool_(False)))

    chunk = tot_v[pl.ds(jstar * L, L)]
    rev = lax.rev(chunk, (0,))
    cs = plsc.cumsum(rev)
    hit = (acc + cs) >= k_b
    nhit = plsc.all_reduce_population_count(hit)
    ffs = jnp.int32(L) - nhit
    sel = (lane == ffs).astype(jnp.int32)
    cs_at = jnp.sum(cs * sel)
    cnt_at = jnp.sum(rev * sel)
    ffs_s = jnp.max(ffs)
    bstar = jstar * L + (L - 1) - ffs_s
    rank = k_b - acc - cs_at + cnt_at
    return bstar, rank


def _make_select(B, HW, k):
    CHUNK = HW // NS
    NV = CHUNK // L
    mesh = plsc.VectorSubcoreMesh(core_axis_name="c", subcore_axis_name="s")

    @functools.partial(
        pl.kernel, mesh=mesh,
        compiler_params=pltpu.CompilerParams(needs_layout_passes=False),
        out_type=jax.ShapeDtypeStruct((B, L), jnp.int32),
        scratch_types=[
            pltpu.VMEM((CHUNK,), jnp.int32),
            pltpu.VMEM((2048,), jnp.int32),
            pltpu.VMEM((2048,), jnp.int32),
            pltpu.VMEM((128,), jnp.int32),
            pltpu.VMEM((128,), jnp.int32),
            pltpu.VMEM((L,), jnp.int32),
            pltpu.VMEM_SHARED((NS, 2048), jnp.int32),
            pltpu.VMEM_SHARED((2048,), jnp.int32),
        ],
    )
    def select(e_hbm, out_hbm, e_v, hist_v, tot_v, row_v, slice_v, thr_v,
               sh_hist, sh_tot):
        cid = lax.axis_index("c")
        sid = lax.axis_index("s")
        ones = jnp.ones((L,), jnp.int32)
        zeros = jnp.zeros((L,), jnp.int32)

        for bi in range(B // 2):
            b = bi * (B // 2) + cid
            pltpu.sync_copy(e_hbm.at[b, pl.ds(sid * CHUNK, CHUNK)], e_v)

            k_b = jnp.int32(k)
            prefix = jnp.int32(0)
            for (shift, bits, pshift) in ((20, 11, 31), (9, 11, 20), (0, 9, 9)):
                nbins = 1 << bits
                nsl = nbins // NS

                def zero_body(j, _):
                    hist_v[pl.ds(j * L, L)] = zeros
                    return 0
                lax.fori_loop(0, nbins // L, zero_body, 0)

                def scan_body(i, _):
                    for u in range(4):
                        v = e_v[pl.ds((i * 4 + u) * L, L)]
                        bins = lax.shift_right_logical(v, shift) & (nbins - 1)
                        if pshift >= 31:
                            m = v >= 0
                        else:
                            m = lax.shift_right_logical(v, pshift) == prefix
                        plsc.addupdate_scatter(hist_v, [bins], ones, mask=m)
                    return 0
                lax.fori_loop(0, NV // 4, scan_body, 0)

                pltpu.sync_copy(hist_v, sh_hist.at[sid])
                plsc.subcore_barrier()

                def agg_init(j, _):
                    slice_v[pl.ds(j * L, L)] = zeros
                    return 0
                lax.fori_loop(0, nsl // L, agg_init, 0)

                def agg_body(t, _):
                    pltpu.sync_copy(sh_hist.at[t, pl.ds(sid * nsl, nsl)],
                                    row_v.at[pl.ds(0, nsl)])

                    def add_body(j, _):
                        slice_v[pl.ds(j * L, L)] = (
                            slice_v[pl.ds(j * L, L)] + row_v[pl.ds(j * L, L)])
                        return 0
                    lax.fori_loop(0, nsl // L, add_body, 0)
                    return 0
                lax.fori_loop(0, NS, agg_body, 0)

                pltpu.sync_copy(slice_v.at[pl.ds(0, nsl)],
                                sh_tot.at[pl.ds(sid * nsl, nsl)])
                plsc.subcore_barrier()
                pltpu.sync_copy(sh_tot.at[pl.ds(0, nbins)],
                                tot_v.at[pl.ds(0, nbins)])

                bstar, rank = _select_bin(tot_v, nbins, k_b)
                if pshift >= 31:
                    prefix = bstar
                else:
                    prefix = (prefix << bits) | bstar
                k_b = rank
                plsc.subcore_barrier()

            @pl.when(sid == 0)
            def _():
                thr_v[...] = jnp.broadcast_to(prefix, (L,)).astype(jnp.int32)
                pltpu.sync_copy(thr_v, out_hbm.at[b])

    return select


@jax.jit
def kernel(x):
    B, C, H, W = x.shape
    HW = H * W
    k = int(0.5 * HW)
    nblk = H // HB

    select2 = _make_select(2, HW, k)
    thrs = []
    energies = []
    for half in range(2):
        off = half * (B // 2)
        e_half = pl.pallas_call(
            _energy_kernel,
            grid=(B // 2, H // HBE),
            in_specs=[pl.BlockSpec((1, C, HBE, W),
                                   lambda b, p, off=off: (b + off, 0, p, 0))],
            out_specs=pl.BlockSpec((1, 1, HBE, W), lambda b, p: (b, 0, p, 0)),
            out_shape=jax.ShapeDtypeStruct((B // 2, 1, H, W), jnp.int32),
        )(x)
        energies.append(e_half)
        thrs.append(select2(e_half.reshape(B // 2, HW)))

    out = None
    for half in range(2):
        off = half * (B // 2)
        in_specs = [
            pl.BlockSpec(memory_space=pltpu.SMEM),
            pl.BlockSpec((1, 1, HB, W), lambda b, p: (b, 0, p, 0)),
            pl.BlockSpec((1, C, HB, W), lambda b, p, off=off: (b + off, 0, p, 0)),
        ]
        operands = [thrs[half], energies[half], x]
        kwargs = {}
        if half == 1:
            in_specs.append(pl.BlockSpec(memory_space=pl.ANY))
            operands.append(out)
            kwargs["input_output_aliases"] = {3: 0}
        out = pl.pallas_call(
            _mask_kernel,
            grid=(B // 2, nblk),
            in_specs=in_specs,
            out_specs=pl.BlockSpec((1, C, HB, W),
                                   lambda b, p, off=off: (b + off, 0, p, 0)),
            out_shape=jax.ShapeDtypeStruct((B, C, H, W), jnp.float32),
            **kwargs,
        )(*operands)

    return out


# speedup vs baseline: 1.0200x; 1.0200x over previous
import functools

import jax
import jax.numpy as jnp
from jax import lax
from jax.experimental import pallas as pl
from jax.experimental.pallas import tpu as pltpu
from jax.experimental.pallas import tpu_sc as plsc

HB = 64
HBE = 64

NS = 16
L = 16


def _energy_kernel(x_ref, o_ref):
    x = x_ref[...]
    e2 = jnp.sum(x * x, axis=1)[:, None]
    o_ref[...] = lax.bitcast_convert_type(e2, jnp.int32)


def _mask_kernel(thr_ref, e_blk_ref, x_ref, *rest):
    o_ref = rest[-1]
    b = pl.program_id(0)
    thr = thr_ref[b, 0]
    mask = (e_blk_ref[...] >= thr).astype(jnp.float32)
    o_ref[...] = x_ref[...] * mask


def _select_bin(tot_v, nbins, k_b):
    nch = nbins // L
    lane = lax.iota(jnp.int32, L)

    def sum_body(jj, carry):
        acc, jstar, done = carry
        j = nch - 1 - jj
        s = jnp.sum(tot_v[pl.ds(j * L, L)])
        crossing = jnp.logical_and(jnp.logical_not(done), (acc + s) >= k_b)
        jstar = jnp.where(crossing, j, jstar)
        done = jnp.logical_or(done, crossing)
        acc = acc + jnp.where(done, jnp.int32(0), s)
        return acc, jstar, done

    acc, jstar, _ = lax.fori_loop(
        0, nch, sum_body, (jnp.int32(0), jnp.int32(0), jnp.bool_(False)))

    chunk = tot_v[pl.ds(jstar * L, L)]
    rev = lax.rev(chunk, (0,))
    cs = plsc.cumsum(rev)
    hit = (acc + cs) >= k_b
    nhit = plsc.all_reduce_population_count(hit)
    ffs = jnp.int32(L) - nhit
    sel = (lane == ffs).astype(jnp.int32)
    cs_at = jnp.sum(cs * sel)
    cnt_at = jnp.sum(rev * sel)
    ffs_s = jnp.max(ffs)
    bstar = jstar * L + (L - 1) - ffs_s
    rank = k_b - acc - cs_at + cnt_at
    return bstar, rank


def _make_select(B, HW, k):
    CHUNK = HW // NS
    NV = CHUNK // L
    mesh = plsc.VectorSubcoreMesh(core_axis_name="c", subcore_axis_name="s")

    @functools.partial(
        pl.kernel, mesh=mesh,
        compiler_params=pltpu.CompilerParams(needs_layout_passes=False),
        out_type=jax.ShapeDtypeStruct((B, L), jnp.int32),
        scratch_types=[
            pltpu.VMEM((CHUNK,), jnp.int32),
            pltpu.VMEM((2048,), jnp.int32),
            pltpu.VMEM((2048,), jnp.int32),
            pltpu.VMEM((128,), jnp.int32),
            pltpu.VMEM((128,), jnp.int32),
            pltpu.VMEM((L,), jnp.int32),
            pltpu.VMEM_SHARED((NS, 2048), jnp.int32),
            pltpu.VMEM_SHARED((2048,), jnp.int32),
        ],
    )
    def select(e_hbm, out_hbm, e_v, hist_v, tot_v, row_v, slice_v, thr_v,
               sh_hist, sh_tot):
        cid = lax.axis_index("c")
        sid = lax.axis_index("s")
        ones = jnp.ones((L,), jnp.int32)
        zeros = jnp.zeros((L,), jnp.int32)

        for bi in range(B // 2):
            b = bi * (B // 2) + cid
            pltpu.sync_copy(e_hbm.at[b, pl.ds(sid * CHUNK, CHUNK)], e_v)

            k_b = jnp.int32(k)
            prefix = jnp.int32(0)
            for (shift, bits, pshift) in ((20, 11, 31), (9, 11, 20), (0, 9, 9)):
                nbins = 1 << bits
                nsl = nbins // NS

                def zero_body(j, _):
                    hist_v[pl.ds(j * L, L)] = zeros
                    return 0
                lax.fori_loop(0, nbins // L, zero_body, 0)

                def scan_body(i, _):
                    for u in range(4):
                        v = e_v[pl.ds((i * 4 + u) * L, L)]
                        bins = lax.shift_right_logical(v, shift) & (nbins - 1)
                        if pshift >= 31:
                            m = v >= 0
                        else:
                            m = lax.shift_right_logical(v, pshift) == prefix
                        plsc.addupdate_scatter(hist_v, [bins], ones, mask=m)
                    return 0
                lax.fori_loop(0, NV // 4, scan_body, 0)

                pltpu.sync_copy(hist_v, sh_hist.at[sid])
                plsc.subcore_barrier()

                def agg_init(j, _):
                    slice_v[pl.ds(j * L, L)] = zeros
                    return 0
                lax.fori_loop(0, nsl // L, agg_init, 0)

                def agg_body(t, _):
                    pltpu.sync_copy(sh_hist.at[t, pl.ds(sid * nsl, nsl)],
                                    row_v.at[pl.ds(0, nsl)])

                    def add_body(j, _):
                        slice_v[pl.ds(j * L, L)] = (
                            slice_v[pl.ds(j * L, L)] + row_v[pl.ds(j * L, L)])
                        return 0
                    lax.fori_loop(0, nsl // L, add_body, 0)
                    return 0
                lax.fori_loop(0, NS, agg_body, 0)

                pltpu.sync_copy(slice_v.at[pl.ds(0, nsl)],
                                sh_tot.at[pl.ds(sid * nsl, nsl)])
                plsc.subcore_barrier()
                pltpu.sync_copy(sh_tot.at[pl.ds(0, nbins)],
                                tot_v.at[pl.ds(0, nbins)])

                bstar, rank = _select_bin(tot_v, nbins, k_b)
                if pshift >= 31:
                    prefix = bstar
                else:
                    prefix = (prefix << bits) | bstar
                k_b = rank
                plsc.subcore_barrier()

            @pl.when(sid == 0)
            def _():
                thr_v[...] = jnp.broadcast_to(prefix, (L,)).astype(jnp.int32)
                pltpu.sync_copy(thr_v, out_hbm.at[b])

    return select


@jax.jit
def kernel(x):
    B, C, H, W = x.shape
    HW = H * W
    k = int(0.5 * HW)
    nblk = H // HB

    select2 = _make_select(2, HW, k)
    thrs = []
    energies = []
    for half in range(2):
        off = half * (B // 2)
        e_half = pl.pallas_call(
            _energy_kernel,
            grid=(B // 2, H // HBE),
            in_specs=[pl.BlockSpec((1, C, HBE, W),
                                   lambda b, p, off=off: (b + off, 0, p, 0))],
            out_specs=pl.BlockSpec((1, 1, HBE, W), lambda b, p: (b, 0, p, 0)),
            out_shape=jax.ShapeDtypeStruct((B // 2, 1, H, W), jnp.int32),
        )(x)
        energies.append(e_half)
        thrs.append(select2(e_half.reshape(B // 2, HW)))

    out = None
    for half in range(2):
        off = half * (B // 2)
        in_specs = [
            pl.BlockSpec(memory_space=pltpu.SMEM),
            pl.BlockSpec((1, 1, HB, W), lambda b, p: (b, 0, p, 0)),
            pl.BlockSpec((1, C, HB, W), lambda b, p, off=off: (b + off, 0, p, 0)),
        ]
        operands = [thrs[half], energies[half], x]
        kwargs = {}
        if half == 1:
            in_specs.append(pl.BlockSpec(memory_space=pl.ANY))
            operands.append(out)
            kwargs["input_output_aliases"] = {3: 0}
        out = pl.pallas_call(
            _mask_kernel,
            grid=(B // 2, nblk),
            in_specs=in_specs,
            out_specs=pl.BlockSpec((1, C, HB, W),
                                   lambda b, p, off=off: (b + off, 0, p, 0)),
            out_shape=jax.ShapeDtypeStruct((B, C, H, W), jnp.float32),
            **kwargs,
        )(*operands)

    return out
